# baseline (speedup 1.0000x reference)
import jax
import jax.numpy as jnp
from jax import lax
from jax.experimental import pallas as pl
from jax.experimental.pallas import tpu as pltpu

_G = 16
_P0 = 2048
_BIG = 3e38
_HI = lax.Precision.HIGHEST
_LV1 = (2048, 512, 16, 0.1, 6, 8, 8)
_LV2 = (512, 128, 32, 0.2, 8, 12, 16)
_LV3 = (128, 32, 64, 0.4, 16, 24, 32)


def _fps(pxyz, cen_ref, mind_ref, S):
    G, P = pxyz[0].shape
    lane = lax.broadcasted_iota(jnp.int32, (G, P), 1)
    q0 = [c[:, 0:1] for c in pxyz]
    cen_ref[:, 0:1, :] = jnp.concatenate(q0, axis=1).reshape(G, 1, 3)
    mind_ref[:, 0:P] = ((pxyz[0] - q0[0]) ** 2 + (pxyz[1] - q0[1]) ** 2
                        + (pxyz[2] - q0[2]) ** 2)

    def body(i, _):
        mind = mind_ref[:, 0:P]
        m = jnp.max(mind, axis=1, keepdims=True)
        nxt = jnp.min(jnp.where(mind == m, lane, P), axis=1, keepdims=True)
        sel = lane == nxt
        q = [jnp.sum(jnp.where(sel, c, 0.0), axis=1, keepdims=True)
             for c in pxyz]
        cen_ref[:, pl.ds(i, 1), :] = jnp.concatenate(q, axis=1).reshape(G, 1, 3)
        d = ((pxyz[0] - q[0]) ** 2 + (pxyz[1] - q[1]) ** 2
             + (pxyz[2] - q[2]) ** 2)
        mind_ref[:, 0:P] = jnp.minimum(mind, d)
        return 0

    lax.fori_loop(1, S, body, 0)


def _conv1(pspl_ref, cen_ref, src_ref, mk_ref, dst_ref, wts):
    P, S, K, r, Cin, H, O = _LV1
    W1, b1, g1, bn1, W2, b2 = wts
    r2 = jnp.float32(r * r)
    laneP = lax.broadcasted_iota(jnp.int32, (S, P), 1)

    def gbody(g, _):
        srcg = jnp.reshape(src_ref[pl.ds(g, 1), :, :], (P, Cin + 3))
        srcxb = srcg[:, 0:Cin].astype(jnp.bfloat16).astype(jnp.float32)
        srcp = srcg[:, Cin:Cin + 3]
        cend = jnp.reshape(cen_ref[pl.ds(g, 1), :, :], (S, 3))
        d2 = jnp.zeros((S, P), jnp.float32)
        for c in range(3):
            cc = jnp.reshape(cen_ref[pl.ds(g, 1), :, c:c + 1], (S, 1))
            pc = jnp.reshape(pspl_ref[c:c + 1, pl.ds(g, 1), :], (1, P))
            diff = cc - pc
            d2 = d2 + diff * diff
        inrad = d2 <= r2
        mk_ref[...] = jnp.where(inrad, d2, _BIG)
        cnt = jnp.sum(inrad.astype(jnp.int32), axis=1)
        trip = jnp.minimum(jnp.int32(K), jnp.max(cnt))

        def kbody(_, acc):
            mk = mk_ref[...]
            m = jnp.min(mk, axis=1, keepdims=True)
            idxm = jnp.min(jnp.where(mk == m, laneP, P), axis=1,
                           keepdims=True)
            sel = laneP == idxm
            mk_ref[...] = jnp.where(sel, _BIG, mk)
            self32 = sel.astype(jnp.float32)
            xj = jnp.dot(self32, srcxb,
                         preferred_element_type=jnp.float32)
            posj = jnp.dot(self32, srcp,
                           preferred_element_type=jnp.float32,
                           precision=_HI)
            rel = posj - cend
            e = jnp.concatenate([xj, rel], axis=1)
            h = jnp.dot(e, W1, preferred_element_type=jnp.float32) + b1
            mu = jnp.mean(h, axis=-1, keepdims=True)
            var = jnp.mean((h - mu) ** 2, axis=-1, keepdims=True)
            h = (h - mu) / jnp.sqrt(var + 1e-5) * g1 + bn1
            h = jax.nn.relu(h)
            h2 = jnp.dot(h, W2, preferred_element_type=jnp.float32) + b2
            h2 = jnp.where(m <= r2, h2, -1e30)
            return jnp.maximum(acc, h2)

        acc = lax.fori_loop(0, trip, kbody,
                            jnp.full((S, O), -1e30, jnp.float32))
        xo = jnp.where(acc <= -1e29, 0.0, acc)
        dst_ref[pl.ds(g, 1), :, :] = jnp.reshape(
            jnp.concatenate([xo, cend], axis=1), (1, S, O + 3))
        return 0

    lax.fori_loop(0, _G, gbody, 0)


def _conv_batched(pspl_ref, cen_ref, src_ref, mk_ref, dst_ref, wts, dims):
    P, S, K, r, Cin, H, O = dims
    W1, b1, g1, bn1, W2, b2 = wts
    r2 = jnp.float32(r * r)
    G = _G
    lane3 = lax.broadcasted_iota(jnp.int32, (G, S, P), 2)
    src = src_ref[...]
    srcxb = src[:, :, 0:Cin].astype(jnp.bfloat16).astype(jnp.float32)
    srcp = src[:, :, Cin:Cin + 3]
    cen = cen_ref[...]
    b1b = jnp.reshape(b1, (1, 1, H))
    g1b = jnp.reshape(g1, (1, 1, H))
    bn1b = jnp.reshape(bn1, (1, 1, H))
    b2b = jnp.reshape(b2, (1, 1, O))

    d2 = jnp.zeros((G, S, P), jnp.float32)
    for c in range(3):
        cc = cen[:, :, c:c + 1]
        pc = jnp.transpose(pspl_ref[c:c + 1, :, :], (1, 0, 2))
        diff = cc - pc
        d2 = d2 + diff * diff
    inrad = d2 <= r2
    mk_ref[...] = jnp.where(inrad, d2, _BIG)
    cnt = jnp.sum(inrad.astype(jnp.int32), axis=2)
    trip = jnp.minimum(jnp.int32(K), jnp.max(cnt))

    def kbody(_, acc):
        mk = mk_ref[...]
        m = jnp.min(mk, axis=2, keepdims=True)
        idxm = jnp.min(jnp.where(mk == m, lane3, P), axis=2, keepdims=True)
        sel = lane3 == idxm
        mk_ref[...] = jnp.where(sel, _BIG, mk)
        self32 = sel.astype(jnp.float32)
        xj = lax.dot_general(self32, srcxb, (((2,), (1,)), ((0,), (0,))),
                             preferred_element_type=jnp.float32)
        posj = lax.dot_general(self32, srcp, (((2,), (1,)), ((0,), (0,))),
                               preferred_element_type=jnp.float32,
                               precision=_HI)
        rel = posj - cen
        e = jnp.concatenate([xj, rel], axis=2)
        h = lax.dot_general(e, W1, (((2,), (0,)), ((), ())),
                            preferred_element_type=jnp.float32) + b1b
        mu = jnp.mean(h, axis=2, keepdims=True)
        var = jnp.mean((h - mu) ** 2, axis=2, keepdims=True)
        h = (h - mu) / jnp.sqrt(var + 1e-5) * g1b + bn1b
        h = jax.nn.relu(h)
        h2 = lax.dot_general(h, W2, (((2,), (0,)), ((), ())),
                             preferred_element_type=jnp.float32) + b2b
        h2 = jnp.where(m <= r2, h2, -1e30)
        return jnp.maximum(acc, h2)

    acc = lax.fori_loop(0, trip, kbody,
                        jnp.full((G, S, O), -1e30, jnp.float32))
    xo = jnp.where(acc <= -1e29, 0.0, acc)
    dst_ref[...] = jnp.concatenate([xo, cen], axis=2)


def _body(*refs):
    (pspl_ref, src1_ref,
     w11, b11, g11, bn11, w12, b12,
     w21, b21, g21, bn21, w22, b22,
     w31, b31, g31, bn31, w32, b32,
     w41, b41, g41, bn41, w42, b42,
     wh1, bh1, wh2, bh2,
     out_ref,
     cen1, cen2, cen3, cenl1, cenl2, src2, src3, src4,
     mind, mkA, mkB, mkC) = refs
    pxyz = [jnp.reshape(pspl_ref[c:c + 1, :, :], (_G, _P0)) for c in range(3)]
    _fps(pxyz, cen1, mind, _LV1[1])
    _conv1(pspl_ref, cen1, src1_ref, mkA, src2,
           (w11[...], b11[...], g11[...], bn11[...], w12[...], b12[...]))
    for c in range(3):
        v = jnp.reshape(cen1[:, :, c:c + 1], (_G, 512))
        cenl1[c:c + 1, :, :] = jnp.reshape(v, (1, _G, 512))
    pxyz2 = [jnp.reshape(cenl1[c:c + 1, :, :], (_G, 512)) for c in range(3)]
    _fps(pxyz2, cen2, mind, _LV2[1])
    _conv_batched(cenl1, cen2, src2, mkB, src3,
                  (w21[...], b21[...], g21[...], bn21[...], w22[...],
                   b22[...]), _LV2)
    for c in range(3):
        v = jnp.reshape(cen2[:, :, c:c + 1], (_G, 128))
        cenl2[c:c + 1, :, :] = jnp.reshape(v, (1, _G, 128))
    pxyz3 = [jnp.reshape(cenl2[c:c + 1, :, :], (_G, 128)) for c in range(3)]
    _fps(pxyz3, cen3, mind, _LV3[1])
    _conv_batched(cenl2, cen3, src3, mkC, src4,
                  (w31[...], b31[...], g31[...], bn31[...], w32[...],
                   b32[...]), _LV3)
    s4 = src4[...]
    h = lax.dot_general(s4, w41[...], (((2,), (0,)), ((), ())),
                        preferred_element_type=jnp.float32) + b41[...]
    mu = jnp.mean(h, axis=-1, keepdims=True)
    var = jnp.mean((h - mu) ** 2, axis=-1, keepdims=True)
    h = (h - mu) / jnp.sqrt(var + 1e-5) * g41[...] + bn41[...]
    h = jax.nn.relu(h)
    h = lax.dot_general(h, w42[...], (((2,), (0,)), ((), ())),
                        preferred_element_type=jnp.float32) + b42[...]
    h = jnp.max(h, axis=1)
    h = jnp.dot(h, wh1[...], preferred_element_type=jnp.float32) + bh1[...]
    out_ref[...] = jnp.dot(h, wh2[...],
                           preferred_element_type=jnp.float32) + bh2[...]


def kernel(x, pos, batch, params):
    pos_g = pos.reshape(_G, _P0, 3)
    pspl = pos_g.transpose(2, 0, 1)
    src1 = jnp.concatenate([x, pos], axis=1).reshape(_G, _P0, 9)
    flat = []
    for name in ("sa1", "sa2", "sa3", "sa4"):
        q = params[name]
        flat += [q["W1"], q["b1"].reshape(1, -1), q["g1"].reshape(1, -1),
                 q["bn1"].reshape(1, -1), q["W2"], q["b2"].reshape(1, -1)]
    hd = params["head"]
    flat += [hd["W1"], hd["b1"].reshape(1, -1), hd["W2"], hd["b2"].reshape(1, -1)]

    return pl.pallas_call(
        _body,
        out_shape=jax.ShapeDtypeStruct((_G, 1), jnp.float32),
        scratch_shapes=[
            pltpu.VMEM((_G, 512, 3), jnp.float32),
            pltpu.VMEM((_G, 128, 3), jnp.float32),
            pltpu.VMEM((_G, 32, 3), jnp.float32),
            pltpu.VMEM((3, _G, 512), jnp.float32),
            pltpu.VMEM((3, _G, 128), jnp.float32),
            pltpu.VMEM((_G, 512, 11), jnp.float32),
            pltpu.VMEM((_G, 128, 19), jnp.float32),
            pltpu.VMEM((_G, 32, 35), jnp.float32),
            pltpu.VMEM((_G, _P0), jnp.float32),
            pltpu.VMEM((512, 2048), jnp.float32),
            pltpu.VMEM((_G, 128, 512), jnp.float32),
            pltpu.VMEM((_G, 32, 128), jnp.float32),
        ],
    )(pspl, src1, *flat)


# speedup vs baseline: 1.0696x; 1.0696x over previous
import jax
import jax.numpy as jnp
from jax import lax
from jax.experimental import pallas as pl
from jax.experimental.pallas import tpu as pltpu

_G = 16
_P0 = 2048
_BIG = 3e38
_HI = lax.Precision.HIGHEST
_LV1 = (2048, 512, 16, 0.1, 6, 8, 8)
_LV2 = (512, 128, 32, 0.2, 8, 12, 16)
_LV3 = (128, 32, 64, 0.4, 16, 24, 32)


def _fps(pxyz, cen_ref, mind_ref, S):
    G, P = pxyz[0].shape
    lane = lax.broadcasted_iota(jnp.int32, (G, P), 1)
    q0 = [c[:, 0:1] for c in pxyz]
    cen_ref[:, 0:1, :] = jnp.concatenate(q0, axis=1).reshape(G, 1, 3)
    mind_ref[:, 0:P] = ((pxyz[0] - q0[0]) ** 2 + (pxyz[1] - q0[1]) ** 2
                        + (pxyz[2] - q0[2]) ** 2)

    def body(i, _):
        mind = mind_ref[:, 0:P]
        m = jnp.max(mind, axis=1, keepdims=True)
        nxt = jnp.min(jnp.where(mind == m, lane, P), axis=1, keepdims=True)
        sel = lane == nxt
        q = [jnp.sum(jnp.where(sel, c, 0.0), axis=1, keepdims=True)
             for c in pxyz]
        cen_ref[:, pl.ds(i, 1), :] = jnp.concatenate(q, axis=1).reshape(G, 1, 3)
        d = ((pxyz[0] - q[0]) ** 2 + (pxyz[1] - q[1]) ** 2
             + (pxyz[2] - q[2]) ** 2)
        mind_ref[:, 0:P] = jnp.minimum(mind, d)
        return 0

    lax.fori_loop(1, S, body, 0)


def _conv1(pspl_ref, cen_ref, src_ref, mk_ref, dst_ref, wts):
    P, S, K, r, Cin, H, O = _LV1
    W1, b1, g1, bn1, W2, b2 = wts
    r2 = jnp.float32(r * r)
    laneP = lax.broadcasted_iota(jnp.int32, (S, P), 1)

    def gbody(g, _):
        srcg = jnp.reshape(src_ref[pl.ds(g, 1), :, :], (P, Cin + 3))
        cend = jnp.reshape(cen_ref[pl.ds(g, 1), :, :], (S, 3))
        d2 = jnp.zeros((S, P), jnp.float32)
        for c in range(3):
            cc = jnp.reshape(cen_ref[pl.ds(g, 1), :, c:c + 1], (S, 1))
            pc = jnp.reshape(pspl_ref[c:c + 1, pl.ds(g, 1), :], (1, P))
            diff = cc - pc
            d2 = d2 + diff * diff
        inrad = d2 <= r2
        mk_ref[...] = jnp.where(inrad, d2, _BIG)
        cnt = jnp.sum(inrad.astype(jnp.int32), axis=1)
        trip = jnp.minimum(jnp.int32(K), jnp.max(cnt))

        def kbody(_, acc):
            mk = mk_ref[...]
            m = jnp.min(mk, axis=1, keepdims=True)
            idxm = jnp.min(jnp.where(mk == m, laneP, P), axis=1,
                           keepdims=True)
            sel = laneP == idxm
            mk_ref[...] = jnp.where(sel, _BIG, mk)
            feat = jnp.dot(sel.astype(jnp.float32), srcg,
                           preferred_element_type=jnp.float32,
                           precision=_HI)
            rel = feat[:, Cin:Cin + 3] - cend
            e = jnp.concatenate([feat[:, 0:Cin], rel], axis=1)
            h = jnp.dot(e, W1, preferred_element_type=jnp.float32) + b1
            mu = jnp.mean(h, axis=-1, keepdims=True)
            var = jnp.mean((h - mu) ** 2, axis=-1, keepdims=True)
            h = (h - mu) / jnp.sqrt(var + 1e-5) * g1 + bn1
            h = jax.nn.relu(h)
            h2 = jnp.dot(h, W2, preferred_element_type=jnp.float32) + b2
            h2 = jnp.where(m <= r2, h2, -1e30)
            return jnp.maximum(acc, h2)

        acc = lax.fori_loop(0, trip, kbody,
                            jnp.full((S, O), -1e30, jnp.float32))
        xo = jnp.where(acc <= -1e29, 0.0, acc)
        dst_ref[pl.ds(g, 1), :, :] = jnp.reshape(
            jnp.concatenate([xo, cend], axis=1), (1, S, O + 3))
        return 0

    lax.fori_loop(0, _G, gbody, 0)


def _conv_batched(pspl_ref, cen_ref, src_ref, mk_ref, dst_ref, wts, dims):
    P, S, K, r, Cin, H, O = dims
    W1, b1, g1, bn1, W2, b2 = wts
    r2 = jnp.float32(r * r)
    G = _G
    lane3 = lax.broadcasted_iota(jnp.int32, (G, S, P), 2)
    src = src_ref[...]
    cen = cen_ref[...]
    b1b = jnp.reshape(b1, (1, 1, H))
    g1b = jnp.reshape(g1, (1, 1, H))
    bn1b = jnp.reshape(bn1, (1, 1, H))
    b2b = jnp.reshape(b2, (1, 1, O))

    d2 = jnp.zeros((G, S, P), jnp.float32)
    for c in range(3):
        cc = cen[:, :, c:c + 1]
        pc = jnp.transpose(pspl_ref[c:c + 1, :, :], (1, 0, 2))
        diff = cc - pc
        d2 = d2 + diff * diff
    inrad = d2 <= r2
    mk_ref[...] = jnp.where(inrad, d2, _BIG)
    cnt = jnp.sum(inrad.astype(jnp.int32), axis=2)
    trip = jnp.minimum(jnp.int32(K), jnp.max(cnt))

    def kbody(_, acc):
        mk = mk_ref[...]
        m = jnp.min(mk, axis=2, keepdims=True)
        idxm = jnp.min(jnp.where(mk == m, lane3, P), axis=2, keepdims=True)
        sel = lane3 == idxm
        mk_ref[...] = jnp.where(sel, _BIG, mk)
        feat = lax.dot_general(sel.astype(jnp.float32), src,
                               (((2,), (1,)), ((0,), (0,))),
                               preferred_element_type=jnp.float32,
                               precision=_HI)
        rel = feat[:, :, Cin:Cin + 3] - cen
        e = jnp.concatenate([feat[:, :, 0:Cin], rel], axis=2)
        h = lax.dot_general(e, W1, (((2,), (0,)), ((), ())),
                            preferred_element_type=jnp.float32) + b1b
        mu = jnp.mean(h, axis=2, keepdims=True)
        var = jnp.mean((h - mu) ** 2, axis=2, keepdims=True)
        h = (h - mu) / jnp.sqrt(var + 1e-5) * g1b + bn1b
        h = jax.nn.relu(h)
        h2 = lax.dot_general(h, W2, (((2,), (0,)), ((), ())),
                             preferred_element_type=jnp.float32) + b2b
        h2 = jnp.where(m <= r2, h2, -1e30)
        return jnp.maximum(acc, h2)

    acc = lax.fori_loop(0, trip, kbody,
                        jnp.full((G, S, O), -1e30, jnp.float32))
    xo = jnp.where(acc <= -1e29, 0.0, acc)
    dst_ref[...] = jnp.concatenate([xo, cen], axis=2)


def _body(*refs):
    (pspl_ref, src1_ref,
     w11, b11, g11, bn11, w12, b12,
     w21, b21, g21, bn21, w22, b22,
     w31, b31, g31, bn31, w32, b32,
     w41, b41, g41, bn41, w42, b42,
     wh1, bh1, wh2, bh2,
     out_ref,
     cen1, cen2, cen3, cenl1, cenl2, src2, src3, src4,
     mind, mkA, mkB, mkC) = refs
    pxyz = [jnp.reshape(pspl_ref[c:c + 1, :, :], (_G, _P0)) for c in range(3)]
    _fps(pxyz, cen1, mind, _LV1[1])
    _conv1(pspl_ref, cen1, src1_ref, mkA, src2,
           (w11[...], b11[...], g11[...], bn11[...], w12[...], b12[...]))
    for c in range(3):
        v = jnp.reshape(cen1[:, :, c:c + 1], (_G, 512))
        cenl1[c:c + 1, :, :] = jnp.reshape(v, (1, _G, 512))
    pxyz2 = [jnp.reshape(cenl1[c:c + 1, :, :], (_G, 512)) for c in range(3)]
    _fps(pxyz2, cen2, mind, _LV2[1])
    _conv_batched(cenl1, cen2, src2, mkB, src3,
                  (w21[...], b21[...], g21[...], bn21[...], w22[...],
                   b22[...]), _LV2)
    for c in range(3):
        v = jnp.reshape(cen2[:, :, c:c + 1], (_G, 128))
        cenl2[c:c + 1, :, :] = jnp.reshape(v, (1, _G, 128))
    pxyz3 = [jnp.reshape(cenl2[c:c + 1, :, :], (_G, 128)) for c in range(3)]
    _fps(pxyz3, cen3, mind, _LV3[1])
    _conv_batched(cenl2, cen3, src3, mkC, src4,
                  (w31[...], b31[...], g31[...], bn31[...], w32[...],
                   b32[...]), _LV3)
    s4 = src4[...]
    h = lax.dot_general(s4, w41[...], (((2,), (0,)), ((), ())),
                        preferred_element_type=jnp.float32) + b41[...]
    mu = jnp.mean(h, axis=-1, keepdims=True)
    var = jnp.mean((h - mu) ** 2, axis=-1, keepdims=True)
    h = (h - mu) / jnp.sqrt(var + 1e-5) * g41[...] + bn41[...]
    h = jax.nn.relu(h)
    h = lax.dot_general(h, w42[...], (((2,), (0,)), ((), ())),
                        preferred_element_type=jnp.float32) + b42[...]
    h = jnp.max(h, axis=1)
    h = jnp.dot(h, wh1[...], preferred_element_type=jnp.float32) + bh1[...]
    out_ref[...] = jnp.dot(h, wh2[...],
                           preferred_element_type=jnp.float32) + bh2[...]


def kernel(x, pos, batch, params):
    pos_g = pos.reshape(_G, _P0, 3)
    pspl = pos_g.transpose(2, 0, 1)
    src1 = jnp.concatenate([x, pos], axis=1).reshape(_G, _P0, 9)
    flat = []
    for name in ("sa1", "sa2", "sa3", "sa4"):
        q = params[name]
        flat += [q["W1"], q["b1"].reshape(1, -1), q["g1"].reshape(1, -1),
                 q["bn1"].reshape(1, -1), q["W2"], q["b2"].reshape(1, -1)]
    hd = params["head"]
    flat += [hd["W1"], hd["b1"].reshape(1, -1), hd["W2"], hd["b2"].reshape(1, -1)]

    return pl.pallas_call(
        _body,
        out_shape=jax.ShapeDtypeStruct((_G, 1), jnp.float32),
        scratch_shapes=[
            pltpu.VMEM((_G, 512, 3), jnp.float32),
            pltpu.VMEM((_G, 128, 3), jnp.float32),
            pltpu.VMEM((_G, 32, 3), jnp.float32),
            pltpu.VMEM((3, _G, 512), jnp.float32),
            pltpu.VMEM((3, _G, 128), jnp.float32),
            pltpu.VMEM((_G, 512, 11), jnp.float32),
            pltpu.VMEM((_G, 128, 19), jnp.float32),
            pltpu.VMEM((_G, 32, 35), jnp.float32),
            pltpu.VMEM((_G, _P0), jnp.float32),
            pltpu.VMEM((512, 2048), jnp.float32),
            pltpu.VMEM((_G, 128, 512), jnp.float32),
            pltpu.VMEM((_G, 32, 128), jnp.float32),
        ],
    )(pspl, src1, *flat)


# speedup vs baseline: 1.0783x; 1.0081x over previous
import jax
import jax.numpy as jnp
from jax import lax
from jax.experimental import pallas as pl
from jax.experimental.pallas import tpu as pltpu

_G = 16
_P0 = 2048
_BIG = 3e38
_HI = lax.Precision.HIGHEST
_LV1 = (2048, 512, 16, 0.1, 6, 8, 8)
_LV2 = (512, 128, 32, 0.2, 8, 12, 16)
_LV3 = (128, 32, 64, 0.4, 16, 24, 32)


def _fps(pxyz, cen_ref, mind_ref, S):
    G, P = pxyz[0].shape
    lane = lax.broadcasted_iota(jnp.int32, (G, P), 1)
    q0 = [c[:, 0:1] for c in pxyz]
    cen_ref[:, 0:1, :] = jnp.concatenate(q0, axis=1).reshape(G, 1, 3)
    mind_ref[:, 0:P] = ((pxyz[0] - q0[0]) ** 2 + (pxyz[1] - q0[1]) ** 2
                        + (pxyz[2] - q0[2]) ** 2)

    def body(i, _):
        mind = mind_ref[:, 0:P]
        m = jnp.max(mind, axis=1, keepdims=True)
        nxt = jnp.min(jnp.where(mind == m, lane, P), axis=1, keepdims=True)
        sel = lane == nxt
        q = [jnp.sum(jnp.where(sel, c, 0.0), axis=1, keepdims=True)
             for c in pxyz]
        cen_ref[:, pl.ds(i, 1), :] = jnp.concatenate(q, axis=1).reshape(G, 1, 3)
        d = ((pxyz[0] - q[0]) ** 2 + (pxyz[1] - q[1]) ** 2
             + (pxyz[2] - q[2]) ** 2)
        mind_ref[:, 0:P] = jnp.minimum(mind, d)
        return 0

    lax.fori_loop(1, S, body, 0)


def _conv1(pspl_ref, cen_ref, src_ref, mk_ref, dst_ref, wts):
    P, S, K, r, Cin, H, O = _LV1
    W1, b1, g1, bn1, W2, b2 = wts
    r2 = jnp.float32(r * r)
    laneP = lax.broadcasted_iota(jnp.int32, (S, P), 1)

    def gbody(g, _):
        srcg = jnp.reshape(src_ref[pl.ds(g, 1), :, :], (P, Cin + 3))
        cend = jnp.reshape(cen_ref[pl.ds(g, 1), :, :], (S, 3))
        d2 = jnp.zeros((S, P), jnp.float32)
        for c in range(3):
            cc = jnp.reshape(cen_ref[pl.ds(g, 1), :, c:c + 1], (S, 1))
            pc = jnp.reshape(pspl_ref[c:c + 1, pl.ds(g, 1), :], (1, P))
            diff = cc - pc
            d2 = d2 + diff * diff
        mk_ref[...] = jnp.where(d2 <= r2, d2, _BIG)

        def kbody(_, acc):
            mk = mk_ref[...]
            m = jnp.min(mk, axis=1, keepdims=True)
            idxm = jnp.min(jnp.where(mk == m, laneP, P), axis=1,
                           keepdims=True)
            sel = laneP == idxm
            mk_ref[...] = jnp.where(sel, _BIG, mk)
            feat = jnp.dot(sel.astype(jnp.float32), srcg,
                           preferred_element_type=jnp.float32,
                           precision=_HI)
            rel = feat[:, Cin:Cin + 3] - cend
            e = jnp.concatenate([feat[:, 0:Cin], rel], axis=1)
            h = jnp.dot(e, W1, preferred_element_type=jnp.float32) + b1
            mu = jnp.mean(h, axis=-1, keepdims=True)
            var = jnp.mean((h - mu) ** 2, axis=-1, keepdims=True)
            h = (h - mu) / jnp.sqrt(var + 1e-5) * g1 + bn1
            h = jax.nn.relu(h)
            h2 = jnp.dot(h, W2, preferred_element_type=jnp.float32) + b2
            h2 = jnp.where(m <= r2, h2, -1e30)
            return jnp.maximum(acc, h2)

        acc = lax.fori_loop(0, K, kbody, jnp.full((S, O), -1e30, jnp.float32))
        xo = jnp.where(acc <= -1e29, 0.0, acc)
        dst_ref[pl.ds(g, 1), :, :] = jnp.reshape(
            jnp.concatenate([xo, cend], axis=1), (1, S, O + 3))
        return 0

    lax.fori_loop(0, _G, gbody, 0)


def _conv_batched(pspl_ref, cen_ref, src_ref, mk_ref, dst_ref, wts, dims):
    P, S, K, r, Cin, H, O = dims
    W1, b1, g1, bn1, W2, b2 = wts
    r2 = jnp.float32(r * r)
    G = _G
    lane3 = lax.broadcasted_iota(jnp.int32, (G, S, P), 2)
    src = src_ref[...]
    cen = cen_ref[...]
    b1b = jnp.reshape(b1, (1, 1, H))
    g1b = jnp.reshape(g1, (1, 1, H))
    bn1b = jnp.reshape(bn1, (1, 1, H))
    b2b = jnp.reshape(b2, (1, 1, O))

    d2 = jnp.zeros((G, S, P), jnp.float32)
    for c in range(3):
        cc = cen[:, :, c:c + 1]
        pc = jnp.transpose(pspl_ref[c:c + 1, :, :], (1, 0, 2))
        diff = cc - pc
        d2 = d2 + diff * diff
    inrad = d2 <= r2
    mk_ref[...] = jnp.where(inrad, d2, _BIG)
    cnt = jnp.sum(inrad.astype(jnp.int32), axis=2)
    trip = jnp.minimum(jnp.int32(K), jnp.max(cnt))

    def kbody(_, acc):
        mk = mk_ref[...]
        m = jnp.min(mk, axis=2, keepdims=True)
        idxm = jnp.min(jnp.where(mk == m, lane3, P), axis=2, keepdims=True)
        sel = lane3 == idxm
        mk_ref[...] = jnp.where(sel, _BIG, mk)
        feat = lax.dot_general(sel.astype(jnp.float32), src,
                               (((2,), (1,)), ((0,), (0,))),
                               preferred_element_type=jnp.float32,
                               precision=_HI)
        rel = feat[:, :, Cin:Cin + 3] - cen
        e = jnp.concatenate([feat[:, :, 0:Cin], rel], axis=2)
        h = lax.dot_general(e, W1, (((2,), (0,)), ((), ())),
                            preferred_element_type=jnp.float32) + b1b
        mu = jnp.mean(h, axis=2, keepdims=True)
        var = jnp.mean((h - mu) ** 2, axis=2, keepdims=True)
        h = (h - mu) / jnp.sqrt(var + 1e-5) * g1b + bn1b
        h = jax.nn.relu(h)
        h2 = lax.dot_general(h, W2, (((2,), (0,)), ((), ())),
                             preferred_element_type=jnp.float32) + b2b
        h2 = jnp.where(m <= r2, h2, -1e30)
        return jnp.maximum(acc, h2)

    acc = lax.fori_loop(0, trip, kbody,
                        jnp.full((G, S, O), -1e30, jnp.float32))
    xo = jnp.where(acc <= -1e29, 0.0, acc)
    dst_ref[...] = jnp.concatenate([xo, cen], axis=2)


def _body(*refs):
    (pspl_ref, src1_ref,
     w11, b11, g11, bn11, w12, b12,
     w21, b21, g21, bn21, w22, b22,
     w31, b31, g31, bn31, w32, b32,
     w41, b41, g41, bn41, w42, b42,
     wh1, bh1, wh2, bh2,
     out_ref,
     cen1, cen2, cen3, cenl1, cenl2, src2, src3, src4,
     mind, mkA, mkB, mkC) = refs
    pxyz = [jnp.reshape(pspl_ref[c:c + 1, :, :], (_G, _P0)) for c in range(3)]
    _fps(pxyz, cen1, mind, _LV1[1])
    _conv1(pspl_ref, cen1, src1_ref, mkA, src2,
           (w11[...], b11[...], g11[...], bn11[...], w12[...], b12[...]))
    for c in range(3):
        v = jnp.reshape(cen1[:, :, c:c + 1], (_G, 512))
        cenl1[c:c + 1, :, :] = jnp.reshape(v, (1, _G, 512))
    pxyz2 = [jnp.reshape(cenl1[c:c + 1, :, :], (_G, 512)) for c in range(3)]
    _fps(pxyz2, cen2, mind, _LV2[1])
    _conv_batched(cenl1, cen2, src2, mkB, src3,
                  (w21[...], b21[...], g21[...], bn21[...], w22[...],
                   b22[...]), _LV2)
    for c in range(3):
        v = jnp.reshape(cen2[:, :, c:c + 1], (_G, 128))
        cenl2[c:c + 1, :, :] = jnp.reshape(v, (1, _G, 128))
    pxyz3 = [jnp.reshape(cenl2[c:c + 1, :, :], (_G, 128)) for c in range(3)]
    _fps(pxyz3, cen3, mind, _LV3[1])
    _conv_batched(cenl2, cen3, src3, mkC, src4,
                  (w31[...], b31[...], g31[...], bn31[...], w32[...],
                   b32[...]), _LV3)
    s4 = src4[...]
    h = lax.dot_general(s4, w41[...], (((2,), (0,)), ((), ())),
                        preferred_element_type=jnp.float32) + b41[...]
    mu = jnp.mean(h, axis=-1, keepdims=True)
    var = jnp.mean((h - mu) ** 2, axis=-1, keepdims=True)
    h = (h - mu) / jnp.sqrt(var + 1e-5) * g41[...] + bn41[...]
    h = jax.nn.relu(h)
    h = lax.dot_general(h, w42[...], (((2,), (0,)), ((), ())),
                        preferred_element_type=jnp.float32) + b42[...]
    h = jnp.max(h, axis=1)
    h = jnp.dot(h, wh1[...], preferred_element_type=jnp.float32) + bh1[...]
    out_ref[...] = jnp.dot(h, wh2[...],
                           preferred_element_type=jnp.float32) + bh2[...]


def kernel(x, pos, batch, params):
    pos_g = pos.reshape(_G, _P0, 3)
    pspl = pos_g.transpose(2, 0, 1)
    src1 = jnp.concatenate([x, pos], axis=1).reshape(_G, _P0, 9)
    flat = []
    for name in ("sa1", "sa2", "sa3", "sa4"):
        q = params[name]
        flat += [q["W1"], q["b1"].reshape(1, -1), q["g1"].reshape(1, -1),
                 q["bn1"].reshape(1, -1), q["W2"], q["b2"].reshape(1, -1)]
    hd = params["head"]
    flat += [hd["W1"], hd["b1"].reshape(1, -1), hd["W2"], hd["b2"].reshape(1, -1)]

    return pl.pallas_call(
        _body,
        out_shape=jax.ShapeDtypeStruct((_G, 1), jnp.float32),
        scratch_shapes=[
            pltpu.VMEM((_G, 512, 3), jnp.float32),
            pltpu.VMEM((_G, 128, 3), jnp.float32),
            pltpu.VMEM((_G, 32, 3), jnp.float32),
            pltpu.VMEM((3, _G, 512), jnp.float32),
            pltpu.VMEM((3, _G, 128), jnp.float32),
            pltpu.VMEM((_G, 512, 11), jnp.float32),
            pltpu.VMEM((_G, 128, 19), jnp.float32),
            pltpu.VMEM((_G, 32, 35), jnp.float32),
            pltpu.VMEM((_G, _P0), jnp.float32),
            pltpu.VMEM((512, 2048), jnp.float32),
            pltpu.VMEM((_G, 128, 512), jnp.float32),
            pltpu.VMEM((_G, 32, 128), jnp.float32),
        ],
    )(pspl, src1, *flat)


# speedup vs baseline: 1.1400x; 1.0572x over previous
import jax
import jax.numpy as jnp
from jax import lax
from jax.experimental import pallas as pl
from jax.experimental.pallas import tpu as pltpu

_G = 16
_P0 = 2048
_BIG = 3e38
_HI = lax.Precision.HIGHEST
_LV1 = (2048, 512, 16, 0.1, 6, 8, 8)
_LV2 = (512, 128, 32, 0.2, 8, 12, 16)
_LV3 = (128, 32, 64, 0.4, 16, 24, 32)


def _fps(pxyz, cen_ref, mind_ref, S):
    G, P = pxyz[0].shape
    lane = lax.broadcasted_iota(jnp.int32, (G, P), 1)
    q0 = [c[:, 0:1] for c in pxyz]
    cen_ref[:, 0:1, :] = jnp.concatenate(q0, axis=1).reshape(G, 1, 3)
    mind_ref[:, 0:P] = ((pxyz[0] - q0[0]) ** 2 + (pxyz[1] - q0[1]) ** 2
                        + (pxyz[2] - q0[2]) ** 2)

    def body(i, _):
        mind = mind_ref[:, 0:P]
        m = jnp.max(mind, axis=1, keepdims=True)
        nxt = jnp.min(jnp.where(mind == m, lane, P), axis=1, keepdims=True)
        sel = lane == nxt
        q = [jnp.sum(jnp.where(sel, c, 0.0), axis=1, keepdims=True)
             for c in pxyz]
        cen_ref[:, pl.ds(i, 1), :] = jnp.concatenate(q, axis=1).reshape(G, 1, 3)
        d = ((pxyz[0] - q[0]) ** 2 + (pxyz[1] - q[1]) ** 2
             + (pxyz[2] - q[2]) ** 2)
        mind_ref[:, 0:P] = jnp.minimum(mind, d)
        return 0

    lax.fori_loop(1, S, body, 0)


def _conv1(pspl_ref, cen_ref, src_ref, mk_ref, dst_ref, wts):
    P, S, K, r, Cin, H, O = _LV1
    GB = 2
    W1, b1, g1, bn1, W2, b2 = wts
    r2 = jnp.float32(r * r)
    lane3 = lax.broadcasted_iota(jnp.int32, (GB, S, P), 2)
    b1b = jnp.reshape(b1, (1, 1, H))
    g1b = jnp.reshape(g1, (1, 1, H))
    bn1b = jnp.reshape(bn1, (1, 1, H))
    b2b = jnp.reshape(b2, (1, 1, O))

    def grp(gi, _):
        g0 = gi * GB
        src = src_ref[pl.ds(g0, GB), :, :]
        cen = cen_ref[pl.ds(g0, GB), :, :]
        d2 = jnp.zeros((GB, S, P), jnp.float32)
        for c in range(3):
            cc = cen[:, :, c:c + 1]
            pc = jnp.concatenate(
                [jnp.reshape(pspl_ref[c:c + 1, pl.ds(g0 + i, 1), :],
                             (1, 1, P)) for i in range(GB)], axis=0)
            diff = cc - pc
            d2 = d2 + diff * diff
        mk_ref[...] = jnp.where(d2 <= r2, d2, _BIG)

        def kbody(_, acc):
            mk = mk_ref[...]
            m = jnp.min(mk, axis=2, keepdims=True)
            idxm = jnp.min(jnp.where(mk == m, lane3, P), axis=2,
                           keepdims=True)
            sel = lane3 == idxm
            mk_ref[...] = jnp.where(sel, _BIG, mk)
            feat = lax.dot_general(sel.astype(jnp.float32), src,
                                   (((2,), (1,)), ((0,), (0,))),
                                   preferred_element_type=jnp.float32,
                                   precision=_HI)
            rel = feat[:, :, Cin:Cin + 3] - cen
            e = jnp.concatenate([feat[:, :, 0:Cin], rel], axis=2)
            h = lax.dot_general(e, W1, (((2,), (0,)), ((), ())),
                                preferred_element_type=jnp.float32) + b1b
            mu = jnp.mean(h, axis=2, keepdims=True)
            var = jnp.mean((h - mu) ** 2, axis=2, keepdims=True)
            h = (h - mu) / jnp.sqrt(var + 1e-5) * g1b + bn1b
            h = jax.nn.relu(h)
            h2 = lax.dot_general(h, W2, (((2,), (0,)), ((), ())),
                                 preferred_element_type=jnp.float32) + b2b
            h2 = jnp.where(m <= r2, h2, -1e30)
            return jnp.maximum(acc, h2)

        acc = lax.fori_loop(0, K, kbody,
                            jnp.full((GB, S, O), -1e30, jnp.float32))
        xo = jnp.where(acc <= -1e29, 0.0, acc)
        dst_ref[pl.ds(g0, GB), :, :] = jnp.concatenate([xo, cen], axis=2)
        return 0

    lax.fori_loop(0, _G // GB, grp, 0)


def _conv_batched(pspl_ref, cen_ref, src_ref, mk_ref, dst_ref, wts, dims):
    P, S, K, r, Cin, H, O = dims
    W1, b1, g1, bn1, W2, b2 = wts
    r2 = jnp.float32(r * r)
    G = _G
    lane3 = lax.broadcasted_iota(jnp.int32, (G, S, P), 2)
    src = src_ref[...]
    cen = cen_ref[...]
    b1b = jnp.reshape(b1, (1, 1, H))
    g1b = jnp.reshape(g1, (1, 1, H))
    bn1b = jnp.reshape(bn1, (1, 1, H))
    b2b = jnp.reshape(b2, (1, 1, O))

    d2 = jnp.zeros((G, S, P), jnp.float32)
    for c in range(3):
        cc = cen[:, :, c:c + 1]
        pc = jnp.transpose(pspl_ref[c:c + 1, :, :], (1, 0, 2))
        diff = cc - pc
        d2 = d2 + diff * diff
    inrad = d2 <= r2
    mk_ref[...] = jnp.where(inrad, d2, _BIG)
    cnt = jnp.sum(inrad.astype(jnp.int32), axis=2)
    trip = jnp.minimum(jnp.int32(K), jnp.max(cnt))

    def kbody(_, acc):
        mk = mk_ref[...]
        m = jnp.min(mk, axis=2, keepdims=True)
        idxm = jnp.min(jnp.where(mk == m, lane3, P), axis=2, keepdims=True)
        sel = lane3 == idxm
        mk_ref[...] = jnp.where(sel, _BIG, mk)
        feat = lax.dot_general(sel.astype(jnp.float32), src,
                               (((2,), (1,)), ((0,), (0,))),
                               preferred_element_type=jnp.float32,
                               precision=_HI)
        rel = feat[:, :, Cin:Cin + 3] - cen
        e = jnp.concatenate([feat[:, :, 0:Cin], rel], axis=2)
        h = lax.dot_general(e, W1, (((2,), (0,)), ((), ())),
                            preferred_element_type=jnp.float32) + b1b
        mu = jnp.mean(h, axis=2, keepdims=True)
        var = jnp.mean((h - mu) ** 2, axis=2, keepdims=True)
        h = (h - mu) / jnp.sqrt(var + 1e-5) * g1b + bn1b
        h = jax.nn.relu(h)
        h2 = lax.dot_general(h, W2, (((2,), (0,)), ((), ())),
                             preferred_element_type=jnp.float32) + b2b
        h2 = jnp.where(m <= r2, h2, -1e30)
        return jnp.maximum(acc, h2)

    acc = lax.fori_loop(0, trip, kbody,
                        jnp.full((G, S, O), -1e30, jnp.float32))
    xo = jnp.where(acc <= -1e29, 0.0, acc)
    dst_ref[...] = jnp.concatenate([xo, cen], axis=2)


def _body(*refs):
    (pspl_ref, src1_ref,
     w11, b11, g11, bn11, w12, b12,
     w21, b21, g21, bn21, w22, b22,
     w31, b31, g31, bn31, w32, b32,
     w41, b41, g41, bn41, w42, b42,
     wh1, bh1, wh2, bh2,
     out_ref,
     cen1, cen2, cen3, cenl1, cenl2, src2, src3, src4,
     mind, mkA, mkB, mkC) = refs
    pxyz = [jnp.reshape(pspl_ref[c:c + 1, :, :], (_G, _P0)) for c in range(3)]
    _fps(pxyz, cen1, mind, _LV1[1])
    _conv1(pspl_ref, cen1, src1_ref, mkA, src2,
           (w11[...], b11[...], g11[...], bn11[...], w12[...], b12[...]))
    for c in range(3):
        v = jnp.reshape(cen1[:, :, c:c + 1], (_G, 512))
        cenl1[c:c + 1, :, :] = jnp.reshape(v, (1, _G, 512))
    pxyz2 = [jnp.reshape(cenl1[c:c + 1, :, :], (_G, 512)) for c in range(3)]
    _fps(pxyz2, cen2, mind, _LV2[1])
    _conv_batched(cenl1, cen2, src2, mkB, src3,
                  (w21[...], b21[...], g21[...], bn21[...], w22[...],
                   b22[...]), _LV2)
    for c in range(3):
        v = jnp.reshape(cen2[:, :, c:c + 1], (_G, 128))
        cenl2[c:c + 1, :, :] = jnp.reshape(v, (1, _G, 128))
    pxyz3 = [jnp.reshape(cenl2[c:c + 1, :, :], (_G, 128)) for c in range(3)]
    _fps(pxyz3, cen3, mind, _LV3[1])
    _conv_batched(cenl2, cen3, src3, mkC, src4,
                  (w31[...], b31[...], g31[...], bn31[...], w32[...],
                   b32[...]), _LV3)
    s4 = src4[...]
    h = lax.dot_general(s4, w41[...], (((2,), (0,)), ((), ())),
                        preferred_element_type=jnp.float32) + b41[...]
    mu = jnp.mean(h, axis=-1, keepdims=True)
    var = jnp.mean((h - mu) ** 2, axis=-1, keepdims=True)
    h = (h - mu) / jnp.sqrt(var + 1e-5) * g41[...] + bn41[...]
    h = jax.nn.relu(h)
    h = lax.dot_general(h, w42[...], (((2,), (0,)), ((), ())),
                        preferred_element_type=jnp.float32) + b42[...]
    h = jnp.max(h, axis=1)
    h = jnp.dot(h, wh1[...], preferred_element_type=jnp.float32) + bh1[...]
    out_ref[...] = jnp.dot(h, wh2[...],
                           preferred_element_type=jnp.float32) + bh2[...]


def kernel(x, pos, batch, params):
    pos_g = pos.reshape(_G, _P0, 3)
    pspl = pos_g.transpose(2, 0, 1)
    src1 = jnp.concatenate([x, pos], axis=1).reshape(_G, _P0, 9)
    flat = []
    for name in ("sa1", "sa2", "sa3", "sa4"):
        q = params[name]
        flat += [q["W1"], q["b1"].reshape(1, -1), q["g1"].reshape(1, -1),
                 q["bn1"].reshape(1, -1), q["W2"], q["b2"].reshape(1, -1)]
    hd = params["head"]
    flat += [hd["W1"], hd["b1"].reshape(1, -1), hd["W2"], hd["b2"].reshape(1, -1)]

    return pl.pallas_call(
        _body,
        out_shape=jax.ShapeDtypeStruct((_G, 1), jnp.float32),
        scratch_shapes=[
            pltpu.VMEM((_G, 512, 3), jnp.float32),
            pltpu.VMEM((_G, 128, 3), jnp.float32),
            pltpu.VMEM((_G, 32, 3), jnp.float32),
            pltpu.VMEM((3, _G, 512), jnp.float32),
            pltpu.VMEM((3, _G, 128), jnp.float32),
            pltpu.VMEM((_G, 512, 11), jnp.float32),
            pltpu.VMEM((_G, 128, 19), jnp.float32),
            pltpu.VMEM((_G, 32, 35), jnp.float32),
            pltpu.VMEM((_G, _P0), jnp.float32),
            pltpu.VMEM((2, 512, 2048), jnp.float32),
            pltpu.VMEM((_G, 128, 512), jnp.float32),
            pltpu.VMEM((_G, 32, 128), jnp.float32),
        ],
    )(pspl, src1, *flat)
